# Initial kernel scaffold; baseline (speedup 1.0000x reference)
import jax
import jax.numpy as jnp
from jax.experimental import pallas as pl


def kernel(x, weight, bias):
    raise NotImplementedError("write your pallas kernel here")


# speedup vs baseline: 1.0908x; 1.0908x over previous
import jax
import jax.numpy as jnp
from jax.experimental import pallas as pl
from jax.experimental.pallas import tpu as pltpu


def _conv_kernel(x_ref, w2_ref, w1_ref, b_ref, o_ref):
    _, oh, ow, c = o_ref.shape
    m = oh * ow

    rows0 = x_ref[0, :oh, 0]
    rows1 = x_ref[0, :oh, 1]
    rows2 = x_ref[0, 1:oh + 1, 0]

    acc = jnp.dot(rows0[:, :ow, :].reshape(m, 2 * c), w2_ref[0],
                  preferred_element_type=jnp.float32)
    acc += b_ref[...]
    acc += jnp.dot(rows1[:, :ow, :].reshape(m, 2 * c), w2_ref[1],
                   preferred_element_type=jnp.float32)
    acc += jnp.dot(rows2[:, :ow, :].reshape(m, 2 * c), w2_ref[2],
                   preferred_element_type=jnp.float32)
    acc += jnp.dot(rows0[:, 1:1 + ow, :c].reshape(m, c), w1_ref[0],
                   preferred_element_type=jnp.float32)
    acc += jnp.dot(rows1[:, 1:1 + ow, :c].reshape(m, c), w1_ref[1],
                   preferred_element_type=jnp.float32)
    acc += jnp.dot(rows2[:, 1:1 + ow, :c].reshape(m, c), w1_ref[2],
                   preferred_element_type=jnp.float32)
    o_ref[...] = acc.reshape(1, oh, ow, c)


def kernel(x, weight, bias):
    n, c, h, w = x.shape
    oh = (h - 2) // 2 + 1
    ow = (w - 2) // 2 + 1
    wp = ow + 1
    hp = oh + 1

    xb = jnp.transpose(x.astype(jnp.bfloat16), (0, 2, 3, 1))
    xp = jnp.pad(xb, ((0, 0), (0, 2 * hp - h), (0, 2 * wp - w), (0, 0)))
    xr = xp.reshape(n, hp, 2, wp, 2 * c)

    w_hwio = jnp.transpose(weight, (2, 3, 1, 0)).astype(jnp.bfloat16)
    w2 = jnp.concatenate([w_hwio[:, 0], w_hwio[:, 1]], axis=1)
    w1 = w_hwio[:, 2]
    b2d = bias.reshape(1, c)

    flops = 2 * n * oh * ow * 9 * c * c
    bytes_accessed = xr.size * 2 + n * oh * ow * c * 4 + (w2.size + w1.size) * 2

    out = pl.pallas_call(
        _conv_kernel,
        out_shape=jax.ShapeDtypeStruct((n, oh, ow, c), jnp.float32),
        grid=(n,),
        in_specs=[
            pl.BlockSpec((1, hp, 2, wp, 2 * c), lambda b: (b, 0, 0, 0, 0)),
            pl.BlockSpec((3, 2 * c, c), lambda b: (0, 0, 0)),
            pl.BlockSpec((3, c, c), lambda b: (0, 0, 0)),
            pl.BlockSpec((1, c), lambda b: (0, 0)),
        ],
        out_specs=pl.BlockSpec((1, oh, ow, c), lambda b: (b, 0, 0, 0)),
        compiler_params=pltpu.CompilerParams(
            dimension_semantics=("parallel",),
            vmem_limit_bytes=48 * 1024 * 1024),
        cost_estimate=pl.CostEstimate(
            flops=flops, transcendentals=0, bytes_accessed=bytes_accessed),
    )(xr, w2, w1, b2d)

    return jnp.transpose(out, (0, 3, 1, 2))
